# Initial kernel scaffold; baseline (speedup 1.0000x reference)
import jax
import jax.numpy as jnp
from jax.experimental import pallas as pl


def kernel(X, L, W):
    raise NotImplementedError("write your pallas kernel here")


# speedup vs baseline: 97.5272x; 97.5272x over previous
import functools

import jax
import jax.numpy as jnp
from jax import lax
from jax.experimental import pallas as pl
from jax.experimental.pallas import tpu as pltpu
from jax.experimental.pallas import tpu_sc as plsc

_NC = 2
_NS = 16
_LANES = 16


@functools.lru_cache(maxsize=None)
def _build(B, D, V):
    NW = _NC * _NS
    assert B % (8 * NW) == 0 and D % _LANES == 0
    b_per_w = B // NW
    l_per_s = B // _NS
    mesh = plsc.VectorSubcoreMesh(core_axis_name="c", subcore_axis_name="s")

    @functools.partial(
        pl.kernel,
        mesh=mesh,
        compiler_params=pltpu.CompilerParams(use_tc_tiling_on_sc=False),
        out_type=[
            jax.ShapeDtypeStruct((B, D), jnp.float32),
            jax.ShapeDtypeStruct((_LANES,), jnp.int32),
        ],
        scratch_types=[
            pltpu.VMEM((b_per_w,), jnp.int32),
            pltpu.VMEM((b_per_w, D), jnp.float32),
            pltpu.VMEM((l_per_s,), jnp.int32),
            pltpu.VMEM((_LANES,), jnp.int32),
            pltpu.VMEM((_NS * _LANES,), jnp.int32),
            pltpu.VMEM_SHARED((_NS * _LANES,), jnp.int32),
            pltpu.SemaphoreType.DMA,
        ],
    )
    def k(idx_hbm, l_hbm, table_hbm, out_hbm, bs_hbm,
          idx_v, rows_v, l_v, cnt_v, all_v, shared, sem):
        c = lax.axis_index("c")
        s = lax.axis_index("s")
        wid = s * _NC + c
        base = wid * b_per_w

        pltpu.sync_copy(idx_hbm.at[pl.ds(base, b_per_w)], idx_v)
        gather = pltpu.async_copy(table_hbm.at[idx_v], rows_v, sem)

        pltpu.sync_copy(l_hbm.at[pl.ds(s * l_per_s, l_per_s)], l_v)
        cnt = jnp.zeros((_LANES,), jnp.int32)
        for i in range(l_per_s // _LANES):
            cnt = cnt + jnp.clip(l_v[pl.ds(i * _LANES, _LANES)], 0, 1)
        cnt_v[...] = cnt
        pltpu.sync_copy(cnt_v, shared.at[pl.ds(s * _LANES, _LANES)])

        plsc.subcore_barrier()

        @pl.when(jnp.logical_and(s == 0, c == 0))
        def _emit():
            pltpu.sync_copy(shared, all_v)
            acc = jnp.zeros((_LANES,), jnp.int32)
            for i in range(_NS):
                acc = acc + all_v[pl.ds(i * _LANES, _LANES)]
            total = acc[0]
            for i in range(1, _LANES):
                total = total + acc[i]
            cnt_v[...] = jnp.full((_LANES,), 0, jnp.int32) + total
            pltpu.sync_copy(cnt_v, bs_hbm)

        gather.wait()
        pltpu.sync_copy(rows_v, out_hbm.at[pl.ds(base, b_per_w)])

    return k


def kernel(X, L, W):
    B = X.shape[0]
    V, D = W.shape
    idx = X[:, 0]
    data, bs = _build(B, D, V)(idx, L.astype(jnp.int32), W)
    return data, bs[_LANES - 1:_LANES]


# speedup vs baseline: 104.3201x; 1.0697x over previous
import functools

import jax
import jax.numpy as jnp
from jax import lax
from jax.experimental import pallas as pl
from jax.experimental.pallas import tpu as pltpu
from jax.experimental.pallas import tpu_sc as plsc

_NC = 2
_NS = 16
_LANES = 16


@functools.lru_cache(maxsize=None)
def _build_gather(B, D, V):
    NW = _NC * _NS
    assert B % (8 * NW) == 0 and D % _LANES == 0
    b_per_w = B // NW
    mesh = plsc.VectorSubcoreMesh(core_axis_name="c", subcore_axis_name="s")

    @functools.partial(
        pl.kernel,
        mesh=mesh,
        compiler_params=pltpu.CompilerParams(use_tc_tiling_on_sc=False),
        out_type=jax.ShapeDtypeStruct((B, D), jnp.float32),
        scratch_types=[
            pltpu.VMEM((b_per_w,), jnp.int32),
            pltpu.VMEM((b_per_w, D), jnp.float32),
            pltpu.SemaphoreType.DMA,
        ],
    )
    def k(idx_hbm, table_hbm, out_hbm, idx_v, rows_v, sem):
        c = lax.axis_index("c")
        s = lax.axis_index("s")
        wid = s * _NC + c
        base = wid * b_per_w

        pltpu.sync_copy(idx_hbm.at[pl.ds(base, b_per_w)], idx_v)
        pltpu.async_copy(table_hbm.at[idx_v], rows_v, sem).wait()
        pltpu.sync_copy(rows_v, out_hbm.at[pl.ds(base, b_per_w)])

    return k


def _count_body(l_ref, out_ref):
    out_ref[0] = jnp.sum((l_ref[...] > 0).astype(jnp.int32))


@functools.lru_cache(maxsize=None)
def _build_count(B):
    return pl.pallas_call(
        _count_body,
        out_shape=jax.ShapeDtypeStruct((1,), jnp.int32),
        in_specs=[pl.BlockSpec(memory_space=pltpu.VMEM)],
        out_specs=pl.BlockSpec(memory_space=pltpu.SMEM),
    )


def kernel(X, L, W):
    B = X.shape[0]
    V, D = W.shape
    idx = X[:, 0]
    data = _build_gather(B, D, V)(idx, W)
    bs = _build_count(B)(L.astype(jnp.int32))
    return data, bs
